# Initial kernel scaffold; baseline (speedup 1.0000x reference)
import jax
import jax.numpy as jnp
from jax.experimental import pallas as pl


def kernel(points, params):
    raise NotImplementedError("write your pallas kernel here")


# speedup vs baseline: 12.7414x; 12.7414x over previous
import functools
import math

import jax
import jax.numpy as jnp
from jax import lax
from jax.experimental import pallas as pl
from jax.experimental.pallas import tpu as pltpu
from jax.experimental.pallas import tpu_sc as plsc

EPS = 1e-5
B = 8
N1 = 4096
S1, K1, R1 = 512, 32, 0.2
S2, K2, R2 = 128, 64, 0.4
DPAD = 16


def _fps_body(npoint, n, x_ref, y_ref, z_ref, cx_ref, cy_ref, cz_ref):
    x = x_ref[...]
    y = y_ref[...]
    z = z_ref[...]
    iota = lax.broadcasted_iota(jnp.int32, (B, n), 1)
    siota = lax.broadcasted_iota(jnp.int32, (B, npoint), 1)

    def body(i, carry):
        dist, far, cxa, cya, cza = carry
        sel = iota == far
        cx = jnp.sum(jnp.where(sel, x, 0.0), axis=1, keepdims=True)
        cy = jnp.sum(jnp.where(sel, y, 0.0), axis=1, keepdims=True)
        cz = jnp.sum(jnp.where(sel, z, 0.0), axis=1, keepdims=True)
        slot = siota == i
        cxa = jnp.where(slot, cx, cxa)
        cya = jnp.where(slot, cy, cya)
        cza = jnp.where(slot, cz, cza)
        d = (x - cx) ** 2 + (y - cy) ** 2 + (z - cz) ** 2
        dist = jnp.minimum(dist, d)
        m = jnp.max(dist, axis=1, keepdims=True)
        far = jnp.min(jnp.where(dist == m, iota, n), axis=1, keepdims=True)
        return dist, far, cxa, cya, cza

    dist0 = jnp.full((B, n), 1e10, jnp.float32)
    far0 = jnp.zeros((B, 1), jnp.int32)
    acc0 = jnp.zeros((B, npoint), jnp.float32)
    _, _, cxa, cya, cza = lax.fori_loop(
        0, npoint, body, (dist0, far0, acc0, acc0, acc0))
    cx_ref[...] = cxa
    cy_ref[...] = cya
    cz_ref[...] = cza


def _fps(x, y, z, npoint):
    n = x.shape[1]
    out = jax.ShapeDtypeStruct((B, npoint), jnp.float32)
    return pl.pallas_call(
        functools.partial(_fps_body, npoint, n),
        out_shape=(out, out, out),
    )(x, y, z)


def _bq_body(n, sc, k, r2, x_ref, y_ref, z_ref, cx_ref, cy_ref, cz_ref,
             idx_ref):
    b = pl.program_id(0)
    x = x_ref[0]
    y = y_ref[0]
    z = z_ref[0]
    cx = cx_ref[0]
    cy = cy_ref[0]
    cz = cz_ref[0]
    sq_x = x * x + y * y + z * z
    sq_c = cx * cx + cy * cy + cz * cz
    cmat = jnp.concatenate([cx, cy, cz], axis=1)
    xmat = jnp.concatenate([x, y, z], axis=0)
    dot = jnp.dot(cmat, xmat, preferred_element_type=jnp.float32)
    d = (sq_c + sq_x) - 2.0 * dot
    iota = lax.broadcasted_iota(jnp.int32, (sc, n), 1)
    validn = jnp.where(d <= r2, iota, n)
    first = jnp.min(validn, axis=1, keepdims=True)
    cols = [first]
    prev = first
    for _ in range(1, k):
        nxt = jnp.min(jnp.where(validn > prev, validn, n), axis=1,
                      keepdims=True)
        cols.append(jnp.where(nxt == n, first, nxt))
        prev = nxt
    out = jnp.concatenate(cols, axis=1) + b * n
    idx_ref[0] = out


def _ball_query(x, y, z, cx, cy, cz, nsample, radius, sc=128):
    n = x.shape[1]
    s = cx.shape[1]
    grid = (B, s // sc)
    x3 = x.reshape(B, 1, n)
    c3 = lambda a: a.reshape(B, s, 1)
    cspec = pl.BlockSpec((1, sc, 1), lambda i, j: (i, j, 0))
    xspec = pl.BlockSpec((1, 1, n), lambda i, j: (i, 0, 0))
    return pl.pallas_call(
        functools.partial(_bq_body, n, sc, nsample, radius * radius),
        grid=grid,
        in_specs=[xspec, xspec, xspec, cspec, cspec, cspec],
        out_specs=pl.BlockSpec((1, sc, nsample), lambda i, j: (i, j, 0)),
        out_shape=jax.ShapeDtypeStruct((B, s, nsample), jnp.int32),
    )(x3, y.reshape(B, 1, n), z.reshape(B, 1, n), c3(cx), c3(cy), c3(cz))


_NW = 32
_CH = 128


def _sc_gather(table, idx):
    m = idx.shape[0]
    d = table.shape[1]
    per = m // _NW
    nch = per // _CH
    mesh = plsc.VectorSubcoreMesh(core_axis_name="c", subcore_axis_name="s")

    @functools.partial(
        pl.kernel, mesh=mesh,
        compiler_params=pltpu.CompilerParams(use_tc_tiling_on_sc=False),
        out_type=jax.ShapeDtypeStruct((m, d), jnp.float32),
        scratch_types=[
            pltpu.VMEM((per,), jnp.int32),
            pltpu.VMEM((_CH, d), jnp.float32),
            pltpu.SemaphoreType.DMA,
        ],
    )
    def k(table_hbm, idx_hbm, out_hbm, idx_v, rows_v, sem):
        wid = lax.axis_index("s") * 2 + lax.axis_index("c")
        base = pl.multiple_of(wid * per, 8)
        pltpu.sync_copy(idx_hbm.at[pl.ds(base, per)], idx_v)

        def body(g, carry):
            off = pl.multiple_of(g * _CH, 8)
            pltpu.async_copy(
                table_hbm.at[idx_v.at[pl.ds(off, _CH)]], rows_v, sem).wait()
            pltpu.sync_copy(
                rows_v, out_hbm.at[pl.ds(pl.multiple_of(base + off, 8), _CH)])
            return carry

        lax.fori_loop(0, nch, body, 0)

    return k(table, idx)


def _affine_relu(h, g, _unused, be):
    return jax.nn.relu(g * (h / jnp.sqrt(jnp.float32(1.0 + EPS))) + be)


def _mlp1_body(gc, k, g_ref, c_ref, w1_ref, w2_ref, w3_ref,
               b1_ref, g1_ref, e1_ref, b2_ref, g2_ref, e2_ref,
               b3_ref, g3_ref, e3_ref, out_ref):
    g = g_ref[...]
    q = jnp.dot(c_ref[...], w1_ref[...],
                preferred_element_type=jnp.float32)
    c1 = q.shape[1]
    qe = jnp.broadcast_to(q[:, None, :], (gc, k, c1)).reshape(gc * k, c1)
    h = jnp.dot(g, w1_ref[...], preferred_element_type=jnp.float32) - qe
    h = _affine_relu(h + b1_ref[...], g1_ref[...], None, e1_ref[...])
    h = jnp.dot(h, w2_ref[...], preferred_element_type=jnp.float32)
    h = _affine_relu(h + b2_ref[...], g2_ref[...], None, e2_ref[...])
    h = jnp.dot(h, w3_ref[...], preferred_element_type=jnp.float32)
    h = _affine_relu(h + b3_ref[...], g3_ref[...], None, e3_ref[...])
    c3 = h.shape[1]
    out_ref[...] = jnp.max(h.reshape(gc, k, c3), axis=1)


def _sa_mlp1(g, c, wts, k, gc):
    (w1, w2, w3, b1, g1, e1, b2, g2, e2, b3, g3, e3) = wts
    s = c.shape[0]
    grid = (s // gc,)
    cout = w3.shape[1]
    wspec = lambda a: pl.BlockSpec(a.shape, lambda i: (0,) * a.ndim)
    return pl.pallas_call(
        functools.partial(_mlp1_body, gc, k),
        grid=grid,
        in_specs=[
            pl.BlockSpec((gc * k, DPAD), lambda i: (i, 0)),
            pl.BlockSpec((gc, DPAD), lambda i: (i, 0)),
            wspec(w1), wspec(w2), wspec(w3),
            wspec(b1), wspec(g1), wspec(e1),
            wspec(b2), wspec(g2), wspec(e2),
            wspec(b3), wspec(g3), wspec(e3),
        ],
        out_specs=pl.BlockSpec((gc, cout), lambda i: (i, 0)),
        out_shape=jax.ShapeDtypeStruct((s, cout), jnp.float32),
    )(g, c, w1, w2, w3, b1, g1, e1, b2, g2, e2, b3, g3, e3)


def _mlp2_body(gc, k, gx_ref, gf_ref, c_ref, w1a_ref, w1b_ref, w2_ref, w3_ref,
               b1_ref, g1_ref, e1_ref, b2_ref, g2_ref, e2_ref,
               b3_ref, g3_ref, e3_ref, out_ref):
    q = jnp.dot(c_ref[...], w1a_ref[...],
                preferred_element_type=jnp.float32)
    c1 = q.shape[1]
    qe = jnp.broadcast_to(q[:, None, :], (gc, k, c1)).reshape(gc * k, c1)
    h = (jnp.dot(gx_ref[...], w1a_ref[...], preferred_element_type=jnp.float32)
         + jnp.dot(gf_ref[...], w1b_ref[...],
                   preferred_element_type=jnp.float32) - qe)
    h = _affine_relu(h + b1_ref[...], g1_ref[...], None, e1_ref[...])
    h = jnp.dot(h, w2_ref[...], preferred_element_type=jnp.float32)
    h = _affine_relu(h + b2_ref[...], g2_ref[...], None, e2_ref[...])
    h = jnp.dot(h, w3_ref[...], preferred_element_type=jnp.float32)
    h = _affine_relu(h + b3_ref[...], g3_ref[...], None, e3_ref[...])
    c3 = h.shape[1]
    out_ref[...] = jnp.max(h.reshape(gc, k, c3), axis=1)


def _sa_mlp2(gx, gf, c, wts, k, gc):
    (w1a, w1b, w2, w3, b1, g1, e1, b2, g2, e2, b3, g3, e3) = wts
    s = c.shape[0]
    din = gf.shape[1]
    grid = (s // gc,)
    cout = w3.shape[1]
    wspec = lambda a: pl.BlockSpec(a.shape, lambda i: (0,) * a.ndim)
    return pl.pallas_call(
        functools.partial(_mlp2_body, gc, k),
        grid=grid,
        in_specs=[
            pl.BlockSpec((gc * k, DPAD), lambda i: (i, 0)),
            pl.BlockSpec((gc * k, din), lambda i: (i, 0)),
            pl.BlockSpec((gc, DPAD), lambda i: (i, 0)),
            wspec(w1a), wspec(w1b), wspec(w2), wspec(w3),
            wspec(b1), wspec(g1), wspec(e1),
            wspec(b2), wspec(g2), wspec(e2),
            wspec(b3), wspec(g3), wspec(e3),
        ],
        out_specs=pl.BlockSpec((gc, cout), lambda i: (i, 0)),
        out_shape=jax.ShapeDtypeStruct((s, cout), jnp.float32),
    )(gx, gf, c, w1a, w1b, w2, w3, b1, g1, e1, b2, g2, e2, b3, g3, e3)


def _head_body(xyz_ref, feat_ref, w1a_ref, w1b_ref, w2_ref, w3_ref,
               b1_ref, g1_ref, e1_ref, b2_ref, g2_ref, e2_ref,
               b3_ref, g3_ref, e3_ref,
               fc1_ref, fb1_ref, n1g_ref, n1b_ref,
               fc2_ref, fb2_ref, n2g_ref, n2b_ref,
               rw_ref, rb_ref, sw_ref, sb_ref, rot_ref, shf_ref):
    h = (jnp.dot(xyz_ref[...], w1a_ref[...], preferred_element_type=jnp.float32)
         + jnp.dot(feat_ref[...], w1b_ref[...],
                   preferred_element_type=jnp.float32))
    h = _affine_relu(h + b1_ref[...], g1_ref[...], None, e1_ref[...])
    h = jnp.dot(h, w2_ref[...], preferred_element_type=jnp.float32)
    h = _affine_relu(h + b2_ref[...], g2_ref[...], None, e2_ref[...])
    h = jnp.dot(h, w3_ref[...], preferred_element_type=jnp.float32)
    h = _affine_relu(h + b3_ref[...], g3_ref[...], None, e3_ref[...])
    c3 = h.shape[1]
    p = jnp.max(h.reshape(B, S2, c3), axis=1)
    f = jnp.dot(p, fc1_ref[...], preferred_element_type=jnp.float32)
    f = _affine_relu(f + fb1_ref[...], n1g_ref[...], None, n1b_ref[...])
    f = jnp.dot(f, fc2_ref[...], preferred_element_type=jnp.float32)
    f = _affine_relu(f + fb2_ref[...], n2g_ref[...], None, n2b_ref[...])
    rot_ref[...] = (jnp.dot(f, rw_ref[...], preferred_element_type=jnp.float32)
                    + rb_ref[...])
    shf_ref[...] = (jnp.dot(f, sw_ref[...], preferred_element_type=jnp.float32)
                    + sb_ref[...])


def _head(xyz, feat, args):
    out = jax.ShapeDtypeStruct((B, 1), jnp.float32)
    return pl.pallas_call(
        _head_body,
        out_shape=(out, out),
    )(xyz, feat, *args)


def _row(v):
    return v.reshape(1, -1)


def _pad16(a):
    return jnp.pad(a, ((0, 0), (0, DPAD - a.shape[1])))


def kernel(points, params):
    x0 = points[:, :, 0]
    y0 = points[:, :, 1]
    z0 = points[:, :, 2]

    cx1, cy1, cz1 = _fps(x0, y0, z0, S1)
    idx1 = _ball_query(x0, y0, z0, cx1, cy1, cz1, K1, R1)
    table1 = _pad16(points.reshape(B * N1, 3))
    c1 = _pad16(jnp.stack([cx1, cy1, cz1], axis=-1).reshape(B * S1, 3))
    g1 = _sc_gather(table1, idx1.reshape(-1))

    w = params
    sa1 = (_pad16(w['sa1_w'][0]).T, w['sa1_w'][1].T, w['sa1_w'][2].T,
           _row(w['sa1_b'][0]), _row(w['sa1_g'][0]), _row(w['sa1_be'][0]),
           _row(w['sa1_b'][1]), _row(w['sa1_g'][1]), _row(w['sa1_be'][1]),
           _row(w['sa1_b'][2]), _row(w['sa1_g'][2]), _row(w['sa1_be'][2]))
    l1 = _sa_mlp1(g1, c1, sa1, K1, 64)

    cx2, cy2, cz2 = _fps(cx1, cy1, cz1, S2)
    idx2 = _ball_query(cx1, cy1, cz1, cx2, cy2, cz2, K2, R2)
    c2 = _pad16(jnp.stack([cx2, cy2, cz2], axis=-1).reshape(B * S2, 3))
    idx2f = idx2.reshape(-1)
    g2x = _sc_gather(c1, idx2f)
    g2f = _sc_gather(l1, idx2f)

    sa2 = (_pad16(w['sa2_w'][0][:, :3]).T, w['sa2_w'][0][:, 3:].T,
           w['sa2_w'][1].T, w['sa2_w'][2].T,
           _row(w['sa2_b'][0]), _row(w['sa2_g'][0]), _row(w['sa2_be'][0]),
           _row(w['sa2_b'][1]), _row(w['sa2_g'][1]), _row(w['sa2_be'][1]),
           _row(w['sa2_b'][2]), _row(w['sa2_g'][2]), _row(w['sa2_be'][2]))
    l2 = _sa_mlp2(g2x, g2f, c2, sa2, K2, 32)

    head_args = (
        _pad16(w['sa3_w'][0][:, :3]).T, w['sa3_w'][0][:, 3:].T,
        w['sa3_w'][1].T, w['sa3_w'][2].T,
        _row(w['sa3_b'][0]), _row(w['sa3_g'][0]), _row(w['sa3_be'][0]),
        _row(w['sa3_b'][1]), _row(w['sa3_g'][1]), _row(w['sa3_be'][1]),
        _row(w['sa3_b'][2]), _row(w['sa3_g'][2]), _row(w['sa3_be'][2]),
        w['fc1_w'].T, _row(w['fc1_b']), _row(w['bn1_g']), _row(w['bn1_b']),
        w['fc2_w'].T, _row(w['fc2_b']), _row(w['bn2_g']), _row(w['bn2_b']),
        w['rot_w'].T, _row(w['rot_b']), w['shift_w'].T, _row(w['shift_b']),
    )
    rot, shift = _head(c2, l2, head_args)
    return rot, shift


# speedup vs baseline: 12.9259x; 1.0145x over previous
import functools
import math

import jax
import jax.numpy as jnp
from jax import lax
from jax.experimental import pallas as pl
from jax.experimental.pallas import tpu as pltpu
from jax.experimental.pallas import tpu_sc as plsc

EPS = 1e-5
B = 8
N1 = 4096
S1, K1, R1 = 512, 32, 0.2
S2, K2, R2 = 128, 64, 0.4
DPAD = 16


def _fps_body(npoint, n, x_ref, y_ref, z_ref, cx_ref, cy_ref, cz_ref):
    x = x_ref[...]
    y = y_ref[...]
    z = z_ref[...]
    iota = lax.broadcasted_iota(jnp.int32, (B, n), 1)
    siota = lax.broadcasted_iota(jnp.int32, (B, npoint), 1)

    def body(i, carry):
        dist, far, cxa, cya, cza = carry
        sel = iota == far
        cx = jnp.sum(jnp.where(sel, x, 0.0), axis=1, keepdims=True)
        cy = jnp.sum(jnp.where(sel, y, 0.0), axis=1, keepdims=True)
        cz = jnp.sum(jnp.where(sel, z, 0.0), axis=1, keepdims=True)
        slot = siota == i
        cxa = jnp.where(slot, cx, cxa)
        cya = jnp.where(slot, cy, cya)
        cza = jnp.where(slot, cz, cza)
        d = (x - cx) ** 2 + (y - cy) ** 2 + (z - cz) ** 2
        dist = jnp.minimum(dist, d)
        m = jnp.max(dist, axis=1, keepdims=True)
        far = jnp.min(jnp.where(dist == m, iota, n), axis=1, keepdims=True)
        return dist, far, cxa, cya, cza

    dist0 = jnp.full((B, n), 1e10, jnp.float32)
    far0 = jnp.zeros((B, 1), jnp.int32)
    acc0 = jnp.zeros((B, npoint), jnp.float32)
    _, _, cxa, cya, cza = lax.fori_loop(
        0, npoint, body, (dist0, far0, acc0, acc0, acc0))
    cx_ref[...] = cxa
    cy_ref[...] = cya
    cz_ref[...] = cza


def _fps(x, y, z, npoint):
    n = x.shape[1]
    out = jax.ShapeDtypeStruct((B, npoint), jnp.float32)
    return pl.pallas_call(
        functools.partial(_fps_body, npoint, n),
        out_shape=(out, out, out),
    )(x, y, z)


def _bq_body(n, sc, k, r2, x_ref, y_ref, z_ref, cx_ref, cy_ref, cz_ref,
             idx_ref):
    b = pl.program_id(0)
    x = x_ref[0]
    y = y_ref[0]
    z = z_ref[0]
    cx = cx_ref[0]
    cy = cy_ref[0]
    cz = cz_ref[0]
    sq_x = x * x + y * y + z * z
    sq_c = cx * cx + cy * cy + cz * cz
    cmat = jnp.concatenate([cx, cy, cz], axis=1)
    xmat = jnp.concatenate([x, y, z], axis=0)
    dot = jnp.dot(cmat, xmat, preferred_element_type=jnp.float32)
    d = (sq_c + sq_x) - 2.0 * dot
    iota = lax.broadcasted_iota(jnp.int32, (sc, n), 1)
    validn = jnp.where(d <= r2, iota, n)
    first = jnp.min(validn, axis=1, keepdims=True)
    cols = [first]
    prev = first
    for _ in range(1, k):
        nxt = jnp.min(jnp.where(validn > prev, validn, n), axis=1,
                      keepdims=True)
        cols.append(jnp.where(nxt == n, first, nxt))
        prev = nxt
    out = jnp.concatenate(cols, axis=1) + b * n
    idx_ref[0] = out


def _ball_query(x, y, z, cx, cy, cz, nsample, radius, sc=128):
    n = x.shape[1]
    s = cx.shape[1]
    grid = (B, s // sc)
    x3 = x.reshape(B, 1, n)
    c3 = lambda a: a.reshape(B, s, 1)
    cspec = pl.BlockSpec((1, sc, 1), lambda i, j: (i, j, 0))
    xspec = pl.BlockSpec((1, 1, n), lambda i, j: (i, 0, 0))
    return pl.pallas_call(
        functools.partial(_bq_body, n, sc, nsample, radius * radius),
        grid=grid,
        in_specs=[xspec, xspec, xspec, cspec, cspec, cspec],
        out_specs=pl.BlockSpec((1, sc, nsample), lambda i, j: (i, j, 0)),
        out_shape=jax.ShapeDtypeStruct((B, s, nsample), jnp.int32),
    )(x3, y.reshape(B, 1, n), z.reshape(B, 1, n), c3(cx), c3(cy), c3(cz))


_NW = 32
_CH = 128


def _sc_gather(tables, idx, burst):
    m = idx.shape[0]
    per = m // _NW
    nq = per // _CH // burst
    ntab = len(tables)
    mesh = plsc.VectorSubcoreMesh(core_axis_name="c", subcore_axis_name="s")
    out_types = tuple(jax.ShapeDtypeStruct((m, t.shape[1]), jnp.float32)
                      for t in tables)
    scratch = ([pltpu.VMEM((per,), jnp.int32)]
               + [pltpu.VMEM((burst * _CH, t.shape[1]), jnp.float32)
                  for t in tables]
               + [pltpu.SemaphoreType.DMA])

    @functools.partial(
        pl.kernel, mesh=mesh,
        compiler_params=pltpu.CompilerParams(use_tc_tiling_on_sc=False),
        out_type=out_types if ntab > 1 else out_types[0],
        scratch_types=scratch,
    )
    def k(*refs):
        tabs = refs[:ntab]
        idx_hbm = refs[ntab]
        outs = refs[ntab + 1:2 * ntab + 1]
        idx_v = refs[2 * ntab + 1]
        rows = refs[2 * ntab + 2:3 * ntab + 2]
        sem = refs[-1]
        wid = lax.axis_index("s") * 2 + lax.axis_index("c")
        base = pl.multiple_of(wid * per, 8)
        pltpu.sync_copy(idx_hbm.at[pl.ds(base, per)], idx_v)

        def body(q, carry):
            offs = [pl.multiple_of((q * burst + bi) * _CH, 8)
                    for bi in range(burst)]
            for t in range(ntab):
                for bi in range(burst):
                    pltpu.async_copy(
                        tabs[t].at[idx_v.at[pl.ds(offs[bi], _CH)]],
                        rows[t].at[pl.ds(bi * _CH, _CH)], sem)
            for t in range(ntab):
                for bi in range(burst):
                    pltpu.make_async_copy(
                        tabs[t].at[pl.ds(0, _CH)],
                        rows[t].at[pl.ds(bi * _CH, _CH)], sem).wait()
            for t in range(ntab):
                for bi in range(burst):
                    pltpu.sync_copy(
                        rows[t].at[pl.ds(bi * _CH, _CH)],
                        outs[t].at[pl.ds(pl.multiple_of(base + offs[bi], 8),
                                         _CH)])
            return carry

        lax.fori_loop(0, nq, body, 0)

    return k(*tables, idx)


def _affine_relu(h, g, _unused, be):
    return jax.nn.relu(g * (h / jnp.sqrt(jnp.float32(1.0 + EPS))) + be)


def _mlp1_body(gc, k, g_ref, c_ref, w1_ref, w2_ref, w3_ref,
               b1_ref, g1_ref, e1_ref, b2_ref, g2_ref, e2_ref,
               b3_ref, g3_ref, e3_ref, out_ref):
    g = g_ref[...]
    c = c_ref[...]
    ce = jnp.broadcast_to(c[:, None, :], (gc, k, DPAD)).reshape(gc * k, DPAD)
    h = jnp.dot(g - ce, w1_ref[...], preferred_element_type=jnp.float32)
    h = _affine_relu(h + b1_ref[...], g1_ref[...], None, e1_ref[...])
    h = jnp.dot(h, w2_ref[...], preferred_element_type=jnp.float32)
    h = _affine_relu(h + b2_ref[...], g2_ref[...], None, e2_ref[...])
    h = jnp.dot(h, w3_ref[...], preferred_element_type=jnp.float32)
    h = _affine_relu(h + b3_ref[...], g3_ref[...], None, e3_ref[...])
    c3 = h.shape[1]
    out_ref[...] = jnp.max(h.reshape(gc, k, c3), axis=1)


def _sa_mlp1(g, c, wts, k, gc):
    (w1, w2, w3, b1, g1, e1, b2, g2, e2, b3, g3, e3) = wts
    s = c.shape[0]
    grid = (s // gc,)
    cout = w3.shape[1]
    wspec = lambda a: pl.BlockSpec(a.shape, lambda i: (0,) * a.ndim)
    return pl.pallas_call(
        functools.partial(_mlp1_body, gc, k),
        grid=grid,
        in_specs=[
            pl.BlockSpec((gc * k, DPAD), lambda i: (i, 0)),
            pl.BlockSpec((gc, DPAD), lambda i: (i, 0)),
            wspec(w1), wspec(w2), wspec(w3),
            wspec(b1), wspec(g1), wspec(e1),
            wspec(b2), wspec(g2), wspec(e2),
            wspec(b3), wspec(g3), wspec(e3),
        ],
        out_specs=pl.BlockSpec((gc, cout), lambda i: (i, 0)),
        out_shape=jax.ShapeDtypeStruct((s, cout), jnp.float32),
    )(g, c, w1, w2, w3, b1, g1, e1, b2, g2, e2, b3, g3, e3)


def _mlp2_body(gc, k, gx_ref, gf_ref, c_ref, w1a_ref, w1b_ref, w2_ref, w3_ref,
               b1_ref, g1_ref, e1_ref, b2_ref, g2_ref, e2_ref,
               b3_ref, g3_ref, e3_ref, out_ref):
    c = c_ref[...]
    ce = jnp.broadcast_to(c[:, None, :], (gc, k, DPAD)).reshape(gc * k, DPAD)
    inp = jnp.concatenate([gx_ref[...] - ce, gf_ref[...]], axis=1)
    w1 = jnp.concatenate([w1a_ref[...], w1b_ref[...]], axis=0)
    h = jnp.dot(inp, w1, preferred_element_type=jnp.float32)
    h = _affine_relu(h + b1_ref[...], g1_ref[...], None, e1_ref[...])
    h = jnp.dot(h, w2_ref[...], preferred_element_type=jnp.float32)
    h = _affine_relu(h + b2_ref[...], g2_ref[...], None, e2_ref[...])
    h = jnp.dot(h, w3_ref[...], preferred_element_type=jnp.float32)
    h = _affine_relu(h + b3_ref[...], g3_ref[...], None, e3_ref[...])
    c3 = h.shape[1]
    out_ref[...] = jnp.max(h.reshape(gc, k, c3), axis=1)


def _sa_mlp2(gx, gf, c, wts, k, gc):
    (w1a, w1b, w2, w3, b1, g1, e1, b2, g2, e2, b3, g3, e3) = wts
    s = c.shape[0]
    din = gf.shape[1]
    grid = (s // gc,)
    cout = w3.shape[1]
    wspec = lambda a: pl.BlockSpec(a.shape, lambda i: (0,) * a.ndim)
    return pl.pallas_call(
        functools.partial(_mlp2_body, gc, k),
        grid=grid,
        in_specs=[
            pl.BlockSpec((gc * k, DPAD), lambda i: (i, 0)),
            pl.BlockSpec((gc * k, din), lambda i: (i, 0)),
            pl.BlockSpec((gc, DPAD), lambda i: (i, 0)),
            wspec(w1a), wspec(w1b), wspec(w2), wspec(w3),
            wspec(b1), wspec(g1), wspec(e1),
            wspec(b2), wspec(g2), wspec(e2),
            wspec(b3), wspec(g3), wspec(e3),
        ],
        out_specs=pl.BlockSpec((gc, cout), lambda i: (i, 0)),
        out_shape=jax.ShapeDtypeStruct((s, cout), jnp.float32),
    )(gx, gf, c, w1a, w1b, w2, w3, b1, g1, e1, b2, g2, e2, b3, g3, e3)


def _head_body(xyz_ref, feat_ref, w1a_ref, w1b_ref, w2_ref, w3_ref,
               b1_ref, g1_ref, e1_ref, b2_ref, g2_ref, e2_ref,
               b3_ref, g3_ref, e3_ref,
               fc1_ref, fb1_ref, n1g_ref, n1b_ref,
               fc2_ref, fb2_ref, n2g_ref, n2b_ref,
               rw_ref, rb_ref, sw_ref, sb_ref, rot_ref, shf_ref):
    inp = jnp.concatenate([xyz_ref[...], feat_ref[...]], axis=1)
    w1 = jnp.concatenate([w1a_ref[...], w1b_ref[...]], axis=0)
    h = jnp.dot(inp, w1, preferred_element_type=jnp.float32)
    h = _affine_relu(h + b1_ref[...], g1_ref[...], None, e1_ref[...])
    h = jnp.dot(h, w2_ref[...], preferred_element_type=jnp.float32)
    h = _affine_relu(h + b2_ref[...], g2_ref[...], None, e2_ref[...])
    h = jnp.dot(h, w3_ref[...], preferred_element_type=jnp.float32)
    h = _affine_relu(h + b3_ref[...], g3_ref[...], None, e3_ref[...])
    c3 = h.shape[1]
    p = jnp.max(h.reshape(B, S2, c3), axis=1)
    f = jnp.dot(p, fc1_ref[...], preferred_element_type=jnp.float32)
    f = _affine_relu(f + fb1_ref[...], n1g_ref[...], None, n1b_ref[...])
    f = jnp.dot(f, fc2_ref[...], preferred_element_type=jnp.float32)
    f = _affine_relu(f + fb2_ref[...], n2g_ref[...], None, n2b_ref[...])
    rot_ref[...] = (jnp.dot(f, rw_ref[...], preferred_element_type=jnp.float32)
                    + rb_ref[...])
    shf_ref[...] = (jnp.dot(f, sw_ref[...], preferred_element_type=jnp.float32)
                    + sb_ref[...])


def _head(xyz, feat, args):
    out = jax.ShapeDtypeStruct((B, 1), jnp.float32)
    return pl.pallas_call(
        _head_body,
        out_shape=(out, out),
    )(xyz, feat, *args)


def _row(v):
    return v.reshape(1, -1)


def _pad16(a):
    return jnp.pad(a, ((0, 0), (0, DPAD - a.shape[1])))


def kernel(points, params):
    x0 = points[:, :, 0]
    y0 = points[:, :, 1]
    z0 = points[:, :, 2]

    cx1, cy1, cz1 = _fps(x0, y0, z0, S1)
    idx1 = _ball_query(x0, y0, z0, cx1, cy1, cz1, K1, R1)
    table1 = _pad16(points.reshape(B * N1, 3))
    c1 = _pad16(jnp.stack([cx1, cy1, cz1], axis=-1).reshape(B * S1, 3))
    g1 = _sc_gather([table1], idx1.reshape(-1), 4)

    w = params
    sa1 = (_pad16(w['sa1_w'][0]).T, w['sa1_w'][1].T, w['sa1_w'][2].T,
           _row(w['sa1_b'][0]), _row(w['sa1_g'][0]), _row(w['sa1_be'][0]),
           _row(w['sa1_b'][1]), _row(w['sa1_g'][1]), _row(w['sa1_be'][1]),
           _row(w['sa1_b'][2]), _row(w['sa1_g'][2]), _row(w['sa1_be'][2]))
    l1 = _sa_mlp1(g1, c1, sa1, K1, 64)

    cx2, cy2, cz2 = _fps(cx1, cy1, cz1, S2)
    idx2 = _ball_query(cx1, cy1, cz1, cx2, cy2, cz2, K2, R2)
    c2 = _pad16(jnp.stack([cx2, cy2, cz2], axis=-1).reshape(B * S2, 3))
    idx2f = idx2.reshape(-1)
    g2x, g2f = _sc_gather([c1, l1], idx2f, 2)

    sa2 = (_pad16(w['sa2_w'][0][:, :3]).T, w['sa2_w'][0][:, 3:].T,
           w['sa2_w'][1].T, w['sa2_w'][2].T,
           _row(w['sa2_b'][0]), _row(w['sa2_g'][0]), _row(w['sa2_be'][0]),
           _row(w['sa2_b'][1]), _row(w['sa2_g'][1]), _row(w['sa2_be'][1]),
           _row(w['sa2_b'][2]), _row(w['sa2_g'][2]), _row(w['sa2_be'][2]))
    l2 = _sa_mlp2(g2x, g2f, c2, sa2, K2, 32)

    head_args = (
        _pad16(w['sa3_w'][0][:, :3]).T, w['sa3_w'][0][:, 3:].T,
        w['sa3_w'][1].T, w['sa3_w'][2].T,
        _row(w['sa3_b'][0]), _row(w['sa3_g'][0]), _row(w['sa3_be'][0]),
        _row(w['sa3_b'][1]), _row(w['sa3_g'][1]), _row(w['sa3_be'][1]),
        _row(w['sa3_b'][2]), _row(w['sa3_g'][2]), _row(w['sa3_be'][2]),
        w['fc1_w'].T, _row(w['fc1_b']), _row(w['bn1_g']), _row(w['bn1_b']),
        w['fc2_w'].T, _row(w['fc2_b']), _row(w['bn2_g']), _row(w['bn2_b']),
        w['rot_w'].T, _row(w['rot_b']), w['shift_w'].T, _row(w['shift_b']),
    )
    rot, shift = _head(c2, l2, head_args)
    return rot, shift
